# baseline (speedup 1.0000x reference)
import functools

import jax
import jax.numpy as jnp
import numpy as np
from jax import lax
from jax.experimental import pallas as pl
from jax.experimental.pallas import tpu as pltpu
from jax.experimental.pallas import tpu_sc as plsc

_OFFSETS = np.array([0, 6, 12, 15, 25, 32, 40, 42, 46, 53, 53], dtype=np.int64)
_PADDING_IDX = 61
_NUM_BAGS = 10
_DIM = 5
_LANES = 16
_N_IDX = 53
_N_ROWS = 100
_NSLOTS = _NUM_BAGS * _DIM
_NCHUNKS = 4
_MAXLEN = int((_OFFSETS[1:] - _OFFSETS[:-1]).max())

_mesh = plsc.VectorSubcoreMesh(
    core_axis_name="c", subcore_axis_name="s", num_cores=1
)


@functools.partial(
    pl.kernel,
    out_type=jax.ShapeDtypeStruct((_NUM_BAGS, _DIM), jnp.float32),
    mesh=_mesh,
    scratch_types=[
        pltpu.VMEM((_N_IDX,), jnp.int32),
        pltpu.VMEM((_N_ROWS, _DIM), jnp.float32),
        pltpu.VMEM((_LANES,), jnp.int32),
        pltpu.VMEM((_NUM_BAGS, _DIM), jnp.float32),
        pltpu.SemaphoreType.DMA,
        pltpu.SemaphoreType.DMA,
        pltpu.SemaphoreType.DMA,
    ],
    compiler_params=pltpu.CompilerParams(
        needs_layout_passes=False,
        disable_bounds_checks=True,
        disable_semaphore_checks=True,
    ),
)
def _bag_mean_sc(inp_hbm, w_hbm, offs_hbm, out_hbm, inp_v, w_v, offs_v, out_v,
                 sem1, sem2, sem3):
    @pl.when(lax.axis_index("s") == 0)
    def _():
        cp1 = pltpu.async_copy(inp_hbm, inp_v, sem1)
        cp2 = pltpu.async_copy(w_hbm, w_v, sem2)
        cp3 = pltpu.async_copy(offs_hbm, offs_v, sem3)
        cp1.wait()
        cp2.wait()
        cp3.wait()
        lane = lax.broadcasted_iota(jnp.int32, (_LANES,), 0)

        def chunk_body(r, _):
            slot = lane + r * _LANES
            bag = jnp.minimum((slot * 13) >> 6, _NUM_BAGS)
            dvec = jnp.minimum(slot - bag * _DIM, _DIM - 1)
            off_vec = plsc.load_gather(offs_v, [bag])
            nxt_vec = plsc.load_gather(offs_v, [jnp.minimum(bag + 1, _NUM_BAGS)])
            len_vec = nxt_vec - off_vec

            def j_body(j, carry):
                acc, cnt = carry
                posv = jnp.minimum(off_vec + j, _N_IDX - 1)
                idx = plsc.load_gather(inp_v, [posv])
                mf = jnp.where(
                    jnp.logical_and(len_vec > j, idx != _PADDING_IDX), 1.0, 0.0
                ).astype(jnp.float32)
                idxc = jnp.minimum(jnp.maximum(idx, 0), _N_ROWS - 1)
                vals = plsc.load_gather(w_v, [idxc, dvec])
                return acc + vals * mf, cnt + mf

            acc, cnt = lax.fori_loop(
                0,
                _MAXLEN,
                j_body,
                (
                    jnp.zeros((_LANES,), jnp.float32),
                    jnp.zeros((_LANES,), jnp.float32),
                ),
            )
            plsc.store_scatter(
                out_v,
                [jnp.minimum(bag, _NUM_BAGS - 1), dvec],
                acc / jnp.maximum(cnt, 1.0),
                mask=slot < _NSLOTS,
            )
            return ()

        lax.fori_loop(0, _NCHUNKS, chunk_body, ())
        pltpu.sync_copy(out_v, out_hbm)


_OFFS_PADDED = np.concatenate(
    [_OFFSETS.astype(np.int32), np.zeros((_LANES - len(_OFFSETS),), np.int32)]
)


def kernel(input, weight):
    return _bag_mean_sc(input, weight, jnp.asarray(_OFFS_PADDED))


# speedup vs baseline: 1.0040x; 1.0040x over previous
import functools

import jax
import jax.numpy as jnp
import numpy as np
from jax import lax
from jax.experimental import pallas as pl
from jax.experimental.pallas import tpu as pltpu
from jax.experimental.pallas import tpu_sc as plsc

_OFFSETS = np.array([0, 6, 12, 15, 25, 32, 40, 42, 46, 53, 53], dtype=np.int64)
_PADDING_IDX = 61
_NUM_BAGS = 10
_DIM = 5
_LANES = 16
_N_IDX = 53
_N_ROWS = 100
_NSLOTS = _NUM_BAGS * _DIM
_NCHUNKS = 4
_MAXLEN = int((_OFFSETS[1:] - _OFFSETS[:-1]).max())

_mesh = plsc.VectorSubcoreMesh(
    core_axis_name="c", subcore_axis_name="s", num_cores=1
)


@functools.partial(
    pl.kernel,
    out_type=jax.ShapeDtypeStruct((_NUM_BAGS, _DIM), jnp.float32),
    mesh=_mesh,
    scratch_types=[
        pltpu.VMEM((_N_IDX,), jnp.int32),
        pltpu.VMEM((_N_ROWS, _DIM), jnp.float32),
        pltpu.VMEM((_LANES,), jnp.int32),
        pltpu.VMEM((_NUM_BAGS, _DIM), jnp.float32),
        pltpu.SemaphoreType.DMA,
        pltpu.SemaphoreType.DMA,
        pltpu.SemaphoreType.DMA,
    ],
    compiler_params=pltpu.CompilerParams(
        needs_layout_passes=False,
        disable_bounds_checks=True,
        disable_semaphore_checks=True,
        use_tc_tiling_on_sc=True,
    ),
)
def _bag_mean_sc(inp_hbm, w_hbm, offs_hbm, out_hbm, inp_v, w_v, offs_v, out_v,
                 sem1, sem2, sem3):
    @pl.when(lax.axis_index("s") == 0)
    def _():
        cp1 = pltpu.async_copy(inp_hbm, inp_v, sem1)
        cp2 = pltpu.async_copy(w_hbm, w_v, sem2)
        cp3 = pltpu.async_copy(offs_hbm, offs_v, sem3)
        cp1.wait()
        cp2.wait()
        cp3.wait()
        lane = lax.broadcasted_iota(jnp.int32, (_LANES,), 0)

        def chunk_body(r, _):
            slot = lane + r * _LANES
            bag = jnp.minimum((slot * 13) >> 6, _NUM_BAGS)
            dvec = jnp.minimum(slot - bag * _DIM, _DIM - 1)
            off_vec = plsc.load_gather(offs_v, [bag])
            nxt_vec = plsc.load_gather(offs_v, [jnp.minimum(bag + 1, _NUM_BAGS)])
            len_vec = nxt_vec - off_vec

            def j_body(j, carry):
                acc, cnt = carry
                posv = jnp.minimum(off_vec + j, _N_IDX - 1)
                idx = plsc.load_gather(inp_v, [posv])
                mf = jnp.where(
                    jnp.logical_and(len_vec > j, idx != _PADDING_IDX), 1.0, 0.0
                ).astype(jnp.float32)
                idxc = jnp.minimum(jnp.maximum(idx, 0), _N_ROWS - 1)
                vals = plsc.load_gather(w_v, [idxc, dvec])
                return acc + vals * mf, cnt + mf

            acc, cnt = lax.fori_loop(
                0,
                _MAXLEN,
                j_body,
                (
                    jnp.zeros((_LANES,), jnp.float32),
                    jnp.zeros((_LANES,), jnp.float32),
                ),
            )
            plsc.store_scatter(
                out_v,
                [jnp.minimum(bag, _NUM_BAGS - 1), dvec],
                acc / jnp.maximum(cnt, 1.0),
                mask=slot < _NSLOTS,
            )
            return ()

        lax.fori_loop(0, _NCHUNKS, chunk_body, ())
        pltpu.sync_copy(out_v, out_hbm)


_OFFS_PADDED = np.concatenate(
    [_OFFSETS.astype(np.int32), np.zeros((_LANES - len(_OFFSETS),), np.int32)]
)


def kernel(input, weight):
    return _bag_mean_sc(input, weight, jnp.asarray(_OFFS_PADDED))
